# baseline (device time: 54362 ns/iter reference)
import jax
import jax.numpy as jnp
from jax import lax
from jax.experimental import pallas as pl
from jax.experimental.pallas import tpu as pltpu

N_DEV = 4
S_PER = 1024
HQ = 8
DH = 128
DM = HQ * DH
S_GLOB = N_DEV * S_PER
HALO = 128
S_BAND = S_PER + 2 * HALO
NG = 32
BQ = 128
BK = BQ + 2 * HALO
PCOLS = DM + 128
SCALE = 0.08838834764831843
NEG = -1e9

PEERS = {0: (1, 2, 3), 1: (0, 2), 2: (0, 1, 3), 3: (0, 2)}


def kernel(x, Wq, K_ext, V_ext, Wo):
    x2 = x.reshape(S_PER, DM)

    def body(x_ref, wq_ref, k_ref, v_ref, wo_ref, out_ref,
             band, gbuf, xb, qs, cst, psend, precv, send_sems, recv_sems):
        my = lax.axis_index("i")
        left = lax.rem(my + N_DEV - 1, N_DEV)
        right = lax.rem(my + 1, N_DEV)

        for h in range(HQ):
            band[pl.ds(HALO, S_PER), h * DH:(h + 1) * DH] = (
                k_ref[0, :, h, :].astype(jnp.bfloat16))
            band[pl.ds(HALO, S_PER), DM + h * DH:DM + (h + 1) * DH] = (
                v_ref[0, :, h, :].astype(jnp.bfloat16))
        xb[...] = x_ref[...].astype(jnp.bfloat16)

        @pl.when(my == 0)
        def _():
            band[pl.ds(0, HALO), :] = jnp.zeros((HALO, 2 * DM), jnp.bfloat16)

        @pl.when(my == N_DEV - 1)
        def _():
            band[pl.ds(S_PER + HALO, HALO), :] = jnp.zeros(
                (HALO, 2 * DM), jnp.bfloat16
            )

        barrier = pltpu.get_barrier_semaphore()
        for srcpos, dsts in PEERS.items():
            @pl.when(my == srcpos)
            def _(dsts=dsts):
                for d in dsts:
                    pl.semaphore_signal(
                        barrier, inc=1, device_id=(d,),
                        device_id_type=pl.DeviceIdType.MESH,
                    )
                for _pad in range(3 - len(dsts)):
                    pl.semaphore_signal(barrier, inc=1)
        pl.semaphore_wait(barrier, 3)

        def halo_to_left():
            return pltpu.make_async_remote_copy(
                src_ref=band.at[pl.ds(HALO, HALO), :],
                dst_ref=band.at[pl.ds(S_PER + HALO, HALO), :],
                send_sem=send_sems.at[0],
                recv_sem=recv_sems.at[1],
                device_id=(left,),
                device_id_type=pl.DeviceIdType.MESH,
            )

        def halo_to_right():
            return pltpu.make_async_remote_copy(
                src_ref=band.at[pl.ds(S_PER, HALO), :],
                dst_ref=band.at[pl.ds(0, HALO), :],
                send_sem=send_sems.at[1],
                recv_sem=recv_sems.at[0],
                device_id=(right,),
                device_id_type=pl.DeviceIdType.MESH,
            )

        def glob_send(d):
            return pltpu.make_async_remote_copy(
                src_ref=gbuf,
                dst_ref=gbuf,
                send_sem=send_sems.at[2 + d],
                recv_sem=recv_sems.at[2],
                device_id=(d,),
                device_id_type=pl.DeviceIdType.MESH,
            )

        def partial_send():
            return pltpu.make_async_remote_copy(
                src_ref=psend,
                dst_ref=precv.at[my - 1],
                send_sem=send_sems.at[2],
                recv_sem=recv_sems.at[2 + my],
                device_id=(0,),
                device_id_type=pl.DeviceIdType.MESH,
            )

        def partial_recv(i):
            return pltpu.make_async_remote_copy(
                src_ref=psend,
                dst_ref=precv.at[i],
                send_sem=send_sems.at[2],
                recv_sem=recv_sems.at[3 + i],
                device_id=(0,),
                device_id_type=pl.DeviceIdType.MESH,
            )

        @pl.when(my == 0)
        def _():
            gbuf[:, 0:2 * DM] = band[pl.ds(HALO, NG), :]
            gbuf[:, 2 * DM:] = xb[0:NG, :]
            for d in (1, 2, 3):
                glob_send(d).start()

        @pl.when(my > 0)
        def _():
            halo_to_left().start()

        @pl.when(my < N_DEV - 1)
        def _():
            halo_to_right().start()

        wqb = wq_ref[...].astype(jnp.bfloat16)
        qs[...] = (jnp.dot(
            xb[...], wqb, preferred_element_type=jnp.float32
        ) * SCALE).astype(jnp.bfloat16)

        @pl.when(my > 0)
        def _():
            glob_send(1).wait_recv()

        q0 = (jnp.dot(
            gbuf[:, 2 * DM:], wqb, preferred_element_type=jnp.float32
        ) * SCALE).astype(jnp.bfloat16)
        ls0 = []
        for h in range(HQ):
            kh = band[pl.ds(HALO, S_PER), h * DH:(h + 1) * DH]
            vh = band[pl.ds(HALO, S_PER), DM + h * DH:DM + (h + 1) * DH]
            s0 = lax.dot_general(
                q0[:, h * DH:(h + 1) * DH], kh, (((1,), (1,)), ((), ())),
                preferred_element_type=jnp.float32,
            )
            e0 = jnp.exp(s0)
            psend[:, h * DH:(h + 1) * DH] = jnp.dot(
                e0.astype(jnp.bfloat16), vh, preferred_element_type=jnp.float32
            )
            ls0.append(jnp.sum(e0, axis=1, keepdims=True))
        psend[:, DM:DM + HQ] = jnp.concatenate(ls0, axis=1)

        @pl.when(my > 0)
        def _():
            partial_send().start()

        @pl.when(my > 0)
        def _():
            halo_to_right().wait_recv()

        @pl.when(my < N_DEV - 1)
        def _():
            halo_to_left().wait_recv()

        qi = lax.broadcasted_iota(jnp.int32, (S_PER, S_BAND), 0)
        kb = lax.broadcasted_iota(jnp.int32, (S_PER, S_BAND), 1)
        kb_g = my * S_PER - HALO + kb
        band_mask = (
            (kb - qi >= 0) & (kb - qi <= 2 * HALO)
            & (kb_g >= 0) & (kb_g < S_GLOB)
        )
        qg = my * S_PER + lax.broadcasted_iota(jnp.int32, (S_PER, NG), 0)
        kg = lax.broadcasted_iota(jnp.int32, (S_PER, NG), 1)
        glob_mask = qg > kg + HALO

        def head_step(h, _):
            qh = qs[:, pl.ds(h * DH, DH)]
            kb_ = band[:, pl.ds(h * DH, DH)]
            vb_ = band[:, pl.ds(DM + h * DH, DH)]
            kg_ = gbuf[:, pl.ds(h * DH, DH)]
            vg_ = gbuf[:, pl.ds(DM + h * DH, DH)]
            sb = lax.dot_general(
                qh, kb_, (((1,), (1,)), ((), ())),
                preferred_element_type=jnp.float32,
            ).astype(jnp.bfloat16)
            pb = jnp.where(band_mask, jnp.exp(sb),
                           jnp.bfloat16(0.0))
            sg = lax.dot_general(
                qh, kg_, (((1,), (1,)), ((), ())),
                preferred_element_type=jnp.float32,
            ).astype(jnp.bfloat16)
            pg = jnp.where(glob_mask, jnp.exp(sg),
                           jnp.bfloat16(0.0))
            l = (jnp.sum(pb.astype(jnp.float32), axis=1, keepdims=True)
                 + jnp.sum(pg.astype(jnp.float32), axis=1, keepdims=True))
            ctx = (
                jnp.dot(pb, vb_, preferred_element_type=jnp.float32)
                + jnp.dot(pg, vg_, preferred_element_type=jnp.float32)
            ) / l
            cst[h] = ctx
            return 0

        lax.fori_loop(0, HQ, head_step, 0)

        @pl.when(my == 0)
        def _():
            for i in range(3):
                partial_recv(i).wait_recv()
            for h in range(HQ):
                num = psend[:, h * DH:(h + 1) * DH]
                den = psend[:, DM + h:DM + h + 1]
                for i in range(3):
                    num = num + precv[i, :, h * DH:(h + 1) * DH]
                    den = den + precv[i, :, DM + h:DM + h + 1]
                cst[h, 0:NG, :] = num / den

        out_ref[...] = jnp.zeros((S_PER, DM), jnp.float32)

        def proj_step(h, _):
            ctx = cst[h].astype(jnp.bfloat16)
            woh = wo_ref[pl.ds(h * DH, DH), :].astype(jnp.bfloat16)
            out_ref[...] = out_ref[...] + jnp.dot(
                ctx, woh, preferred_element_type=jnp.float32
            )
            return 0

        lax.fori_loop(0, HQ, proj_step, 0)

        @pl.when(my > 0)
        def _():
            halo_to_left().wait_send()
            partial_send().wait_send()

        @pl.when(my < N_DEV - 1)
        def _():
            halo_to_right().wait_send()

        @pl.when(my == 0)
        def _():
            for d in (1, 2, 3):
                glob_send(d).wait_send()

    out = pl.pallas_call(
        body,
        out_shape=jax.ShapeDtypeStruct((S_PER, DM), jnp.float32),
        in_specs=[
            pl.BlockSpec(memory_space=pltpu.VMEM),
            pl.BlockSpec(memory_space=pltpu.VMEM),
            pl.BlockSpec(memory_space=pltpu.VMEM),
            pl.BlockSpec(memory_space=pltpu.VMEM),
            pl.BlockSpec(memory_space=pltpu.VMEM),
        ],
        out_specs=pl.BlockSpec(memory_space=pltpu.VMEM),
        scratch_shapes=[
            pltpu.VMEM((S_BAND, 2 * DM), jnp.bfloat16),
            pltpu.VMEM((NG, 2 * DM + DM), jnp.bfloat16),
            pltpu.VMEM((S_PER, DM), jnp.bfloat16),
            pltpu.VMEM((S_PER, DM), jnp.bfloat16),
            pltpu.VMEM((HQ, S_PER, DH), jnp.float32),
            pltpu.VMEM((NG, PCOLS), jnp.float32),
            pltpu.VMEM((3, NG, PCOLS), jnp.float32),
            pltpu.SemaphoreType.DMA((6,)),
            pltpu.SemaphoreType.DMA((6,)),
        ],
        compiler_params=pltpu.CompilerParams(
            collective_id=0, vmem_limit_bytes=100 * 1024 * 1024
        ),
    )(x2, Wq, K_ext, V_ext, Wo)
    return out.reshape(1, S_PER, DM)


# device time: 52645 ns/iter; 1.0326x vs baseline; 1.0326x over previous
import jax
import jax.numpy as jnp
from jax import lax
from jax.experimental import pallas as pl
from jax.experimental.pallas import tpu as pltpu

N_DEV = 4
S_PER = 1024
HQ = 8
DH = 128
DM = HQ * DH
S_GLOB = N_DEV * S_PER
HALO = 128
S_BAND = S_PER + 2 * HALO
NG = 32
BQ = 128
BK = BQ + 2 * HALO
PCOLS = DM + 128
SCALE = 0.08838834764831843
NEG = -1e9

PEERS = {0: (1, 2, 3), 1: (0, 2), 2: (0, 1, 3), 3: (0, 2)}


def kernel(x, Wq, K_ext, V_ext, Wo):
    x2 = x.reshape(S_PER, DM)

    def body(x_ref, wq_ref, k_ref, v_ref, wo_ref, out_ref,
             band, gbuf, xb, qs, cst, psend, precv, send_sems, recv_sems):
        my = lax.axis_index("i")
        left = lax.rem(my + N_DEV - 1, N_DEV)
        right = lax.rem(my + 1, N_DEV)

        for h in range(HQ):
            band[pl.ds(HALO, S_PER), h * DH:(h + 1) * DH] = (
                k_ref[0, :, h, :].astype(jnp.bfloat16))
            band[pl.ds(HALO, S_PER), DM + h * DH:DM + (h + 1) * DH] = (
                v_ref[0, :, h, :].astype(jnp.bfloat16))
        xb[...] = x_ref[...].astype(jnp.bfloat16)

        @pl.when(my == 0)
        def _():
            band[pl.ds(0, HALO), :] = jnp.zeros((HALO, 2 * DM), jnp.bfloat16)

        @pl.when(my == N_DEV - 1)
        def _():
            band[pl.ds(S_PER + HALO, HALO), :] = jnp.zeros(
                (HALO, 2 * DM), jnp.bfloat16
            )

        barrier = pltpu.get_barrier_semaphore()
        for srcpos, dsts in PEERS.items():
            @pl.when(my == srcpos)
            def _(dsts=dsts):
                for d in dsts:
                    pl.semaphore_signal(
                        barrier, inc=1, device_id=(d,),
                        device_id_type=pl.DeviceIdType.MESH,
                    )
                for _pad in range(3 - len(dsts)):
                    pl.semaphore_signal(barrier, inc=1)
        pl.semaphore_wait(barrier, 3)

        def halo_to_left():
            return pltpu.make_async_remote_copy(
                src_ref=band.at[pl.ds(HALO, HALO), :],
                dst_ref=band.at[pl.ds(S_PER + HALO, HALO), :],
                send_sem=send_sems.at[0],
                recv_sem=recv_sems.at[1],
                device_id=(left,),
                device_id_type=pl.DeviceIdType.MESH,
            )

        def halo_to_right():
            return pltpu.make_async_remote_copy(
                src_ref=band.at[pl.ds(S_PER, HALO), :],
                dst_ref=band.at[pl.ds(0, HALO), :],
                send_sem=send_sems.at[1],
                recv_sem=recv_sems.at[0],
                device_id=(right,),
                device_id_type=pl.DeviceIdType.MESH,
            )

        def glob_send(d):
            return pltpu.make_async_remote_copy(
                src_ref=gbuf,
                dst_ref=gbuf,
                send_sem=send_sems.at[2 + d],
                recv_sem=recv_sems.at[2],
                device_id=(d,),
                device_id_type=pl.DeviceIdType.MESH,
            )

        def partial_send():
            return pltpu.make_async_remote_copy(
                src_ref=psend,
                dst_ref=precv.at[my - 1],
                send_sem=send_sems.at[2],
                recv_sem=recv_sems.at[2 + my],
                device_id=(0,),
                device_id_type=pl.DeviceIdType.MESH,
            )

        def partial_recv(i):
            return pltpu.make_async_remote_copy(
                src_ref=psend,
                dst_ref=precv.at[i],
                send_sem=send_sems.at[2],
                recv_sem=recv_sems.at[3 + i],
                device_id=(0,),
                device_id_type=pl.DeviceIdType.MESH,
            )

        @pl.when(my == 0)
        def _():
            gbuf[:, 0:2 * DM] = band[pl.ds(HALO, NG), :]
            gbuf[:, 2 * DM:] = xb[0:NG, :]
            for d in (1, 2, 3):
                glob_send(d).start()

        @pl.when(my > 0)
        def _():
            halo_to_left().start()

        @pl.when(my < N_DEV - 1)
        def _():
            halo_to_right().start()

        wqb = wq_ref[...].astype(jnp.bfloat16)
        qs[...] = (jnp.dot(
            xb[...], wqb, preferred_element_type=jnp.float32
        ) * SCALE).astype(jnp.bfloat16)

        @pl.when(my > 0)
        def _():
            glob_send(1).wait_recv()

        q0 = (jnp.dot(
            gbuf[:, 2 * DM:], wqb, preferred_element_type=jnp.float32
        ) * SCALE).astype(jnp.bfloat16)
        ls0 = []
        for h in range(HQ):
            kh = band[pl.ds(HALO, S_PER), h * DH:(h + 1) * DH]
            vh = band[pl.ds(HALO, S_PER), DM + h * DH:DM + (h + 1) * DH]
            s0 = lax.dot_general(
                q0[:, h * DH:(h + 1) * DH], kh, (((1,), (1,)), ((), ())),
                preferred_element_type=jnp.float32,
            )
            e0 = jnp.exp(s0)
            psend[:, h * DH:(h + 1) * DH] = jnp.dot(
                e0.astype(jnp.bfloat16), vh, preferred_element_type=jnp.float32
            )
            ls0.append(jnp.sum(e0, axis=1, keepdims=True))
        psend[:, DM:DM + HQ] = jnp.concatenate(ls0, axis=1)

        @pl.when(my > 0)
        def _():
            partial_send().start()

        @pl.when(my > 0)
        def _():
            halo_to_right().wait_recv()

        @pl.when(my < N_DEV - 1)
        def _():
            halo_to_left().wait_recv()

        HB = S_PER // 2
        WB = HB + 2 * HALO
        bmasks = []
        gmasks = []
        for j in (0, 1):
            qi = j * HB + lax.broadcasted_iota(jnp.int32, (HB, WB), 0)
            kb = j * HB + lax.broadcasted_iota(jnp.int32, (HB, WB), 1)
            kb_g = my * S_PER - HALO + kb
            bmasks.append(
                (kb - qi >= 0) & (kb - qi <= 2 * HALO)
                & (kb_g >= 0) & (kb_g < S_GLOB)
            )
            qg = (my * S_PER + j * HB
                  + lax.broadcasted_iota(jnp.int32, (HB, NG), 0))
            kg = lax.broadcasted_iota(jnp.int32, (HB, NG), 1)
            gmasks.append(qg > kg + HALO)

        def head_step(h, _):
            kg_ = gbuf[:, pl.ds(h * DH, DH)]
            vg_ = gbuf[:, pl.ds(DM + h * DH, DH)]
            for j in (0, 1):
                qj = qs[pl.ds(j * HB, HB), pl.ds(h * DH, DH)]
                kbj = band[pl.ds(j * HB, WB), pl.ds(h * DH, DH)]
                vbj = band[pl.ds(j * HB, WB), pl.ds(DM + h * DH, DH)]
                sb = lax.dot_general(
                    qj, kbj, (((1,), (1,)), ((), ())),
                    preferred_element_type=jnp.float32,
                ).astype(jnp.bfloat16)
                pb = jnp.where(bmasks[j], jnp.exp(sb), jnp.bfloat16(0.0))
                sg = lax.dot_general(
                    qj, kg_, (((1,), (1,)), ((), ())),
                    preferred_element_type=jnp.float32,
                ).astype(jnp.bfloat16)
                pg = jnp.where(gmasks[j], jnp.exp(sg), jnp.bfloat16(0.0))
                l = (jnp.sum(pb.astype(jnp.float32), axis=1, keepdims=True)
                     + jnp.sum(pg.astype(jnp.float32), axis=1, keepdims=True))
                ctx = (
                    jnp.dot(pb, vbj, preferred_element_type=jnp.float32)
                    + jnp.dot(pg, vg_, preferred_element_type=jnp.float32)
                ) / l
                cst[h, pl.ds(j * HB, HB), :] = ctx
            return 0

        lax.fori_loop(0, HQ, head_step, 0)

        @pl.when(my == 0)
        def _():
            for i in range(3):
                partial_recv(i).wait_recv()
            for h in range(HQ):
                num = psend[:, h * DH:(h + 1) * DH]
                den = psend[:, DM + h:DM + h + 1]
                for i in range(3):
                    num = num + precv[i, :, h * DH:(h + 1) * DH]
                    den = den + precv[i, :, DM + h:DM + h + 1]
                cst[h, 0:NG, :] = num / den

        out_ref[...] = jnp.zeros((S_PER, DM), jnp.float32)

        def proj_step(h, _):
            ctx = cst[h].astype(jnp.bfloat16)
            woh = wo_ref[pl.ds(h * DH, DH), :].astype(jnp.bfloat16)
            out_ref[...] = out_ref[...] + jnp.dot(
                ctx, woh, preferred_element_type=jnp.float32
            )
            return 0

        lax.fori_loop(0, HQ, proj_step, 0)

        @pl.when(my > 0)
        def _():
            halo_to_left().wait_send()
            partial_send().wait_send()

        @pl.when(my < N_DEV - 1)
        def _():
            halo_to_right().wait_send()

        @pl.when(my == 0)
        def _():
            for d in (1, 2, 3):
                glob_send(d).wait_send()

    out = pl.pallas_call(
        body,
        out_shape=jax.ShapeDtypeStruct((S_PER, DM), jnp.float32),
        in_specs=[
            pl.BlockSpec(memory_space=pltpu.VMEM),
            pl.BlockSpec(memory_space=pltpu.VMEM),
            pl.BlockSpec(memory_space=pltpu.VMEM),
            pl.BlockSpec(memory_space=pltpu.VMEM),
            pl.BlockSpec(memory_space=pltpu.VMEM),
        ],
        out_specs=pl.BlockSpec(memory_space=pltpu.VMEM),
        scratch_shapes=[
            pltpu.VMEM((S_BAND, 2 * DM), jnp.bfloat16),
            pltpu.VMEM((NG, 2 * DM + DM), jnp.bfloat16),
            pltpu.VMEM((S_PER, DM), jnp.bfloat16),
            pltpu.VMEM((S_PER, DM), jnp.bfloat16),
            pltpu.VMEM((HQ, S_PER, DH), jnp.float32),
            pltpu.VMEM((NG, PCOLS), jnp.float32),
            pltpu.VMEM((3, NG, PCOLS), jnp.float32),
            pltpu.SemaphoreType.DMA((6,)),
            pltpu.SemaphoreType.DMA((6,)),
        ],
        compiler_params=pltpu.CompilerParams(
            collective_id=0, vmem_limit_bytes=100 * 1024 * 1024
        ),
    )(x2, Wq, K_ext, V_ext, Wo)
    return out.reshape(1, S_PER, DM)


# device time: 48881 ns/iter; 1.1121x vs baseline; 1.0770x over previous
import jax
import jax.numpy as jnp
from jax import lax
from jax.experimental import pallas as pl
from jax.experimental.pallas import tpu as pltpu

N_DEV = 4
S_PER = 1024
HQ = 8
DH = 128
DM = HQ * DH
S_GLOB = N_DEV * S_PER
HALO = 128
S_BAND = S_PER + 2 * HALO
NG = 32
BQ = 128
BK = BQ + 2 * HALO
PCOLS = DM + 128
SCALE = 0.08838834764831843
NEG = -1e9

PEERS = {0: (1, 2, 3), 1: (0, 2), 2: (0, 1, 3), 3: (0, 2)}


def kernel(x, Wq, K_ext, V_ext, Wo):
    x2 = x.reshape(S_PER, DM)

    def body(x_ref, wq_ref, k_ref, v_ref, wo_ref, out_ref,
             band, gbuf, xb, qs, cst, psend, precv, send_sems, recv_sems):
        my = lax.axis_index("i")
        left = lax.rem(my + N_DEV - 1, N_DEV)
        right = lax.rem(my + 1, N_DEV)

        for h in range(HQ):
            band[pl.ds(HALO, S_PER), h * DH:(h + 1) * DH] = (
                k_ref[0, :, h, :].astype(jnp.bfloat16))
            band[pl.ds(HALO, S_PER), DM + h * DH:DM + (h + 1) * DH] = (
                v_ref[0, :, h, :].astype(jnp.bfloat16))
        xb[...] = x_ref[...].astype(jnp.bfloat16)

        @pl.when(my == 0)
        def _():
            band[pl.ds(0, HALO), :] = jnp.zeros((HALO, 2 * DM), jnp.bfloat16)

        @pl.when(my == N_DEV - 1)
        def _():
            band[pl.ds(S_PER + HALO, HALO), :] = jnp.zeros(
                (HALO, 2 * DM), jnp.bfloat16
            )

        barrier = pltpu.get_barrier_semaphore()
        for srcpos, dsts in PEERS.items():
            @pl.when(my == srcpos)
            def _(dsts=dsts):
                for d in dsts:
                    pl.semaphore_signal(
                        barrier, inc=1, device_id=(d,),
                        device_id_type=pl.DeviceIdType.MESH,
                    )
                for _pad in range(3 - len(dsts)):
                    pl.semaphore_signal(barrier, inc=1)
        pl.semaphore_wait(barrier, 3)

        def halo_to_left():
            return pltpu.make_async_remote_copy(
                src_ref=band.at[pl.ds(HALO, HALO), :],
                dst_ref=band.at[pl.ds(S_PER + HALO, HALO), :],
                send_sem=send_sems.at[0],
                recv_sem=recv_sems.at[1],
                device_id=(left,),
                device_id_type=pl.DeviceIdType.MESH,
            )

        def halo_to_right():
            return pltpu.make_async_remote_copy(
                src_ref=band.at[pl.ds(S_PER, HALO), :],
                dst_ref=band.at[pl.ds(0, HALO), :],
                send_sem=send_sems.at[1],
                recv_sem=recv_sems.at[0],
                device_id=(right,),
                device_id_type=pl.DeviceIdType.MESH,
            )

        def glob_send(d):
            return pltpu.make_async_remote_copy(
                src_ref=gbuf,
                dst_ref=gbuf,
                send_sem=send_sems.at[2 + d],
                recv_sem=recv_sems.at[2],
                device_id=(d,),
                device_id_type=pl.DeviceIdType.MESH,
            )

        def partial_send():
            return pltpu.make_async_remote_copy(
                src_ref=psend,
                dst_ref=precv.at[my - 1],
                send_sem=send_sems.at[2],
                recv_sem=recv_sems.at[2 + my],
                device_id=(0,),
                device_id_type=pl.DeviceIdType.MESH,
            )

        def partial_recv(i):
            return pltpu.make_async_remote_copy(
                src_ref=psend,
                dst_ref=precv.at[i],
                send_sem=send_sems.at[2],
                recv_sem=recv_sems.at[3 + i],
                device_id=(0,),
                device_id_type=pl.DeviceIdType.MESH,
            )

        @pl.when(my == 0)
        def _():
            gbuf[:, 0:2 * DM] = band[pl.ds(HALO, NG), :]
            gbuf[:, 2 * DM:] = xb[0:NG, :]
            for d in (1, 2, 3):
                glob_send(d).start()

        @pl.when(my > 0)
        def _():
            halo_to_left().start()

        @pl.when(my < N_DEV - 1)
        def _():
            halo_to_right().start()

        wqb = wq_ref[...].astype(jnp.bfloat16)
        qs[...] = (jnp.dot(
            xb[...], wqb, preferred_element_type=jnp.float32
        ) * SCALE).astype(jnp.bfloat16)

        @pl.when(my > 0)
        def _():
            glob_send(1).wait_recv()

        q0 = (jnp.dot(
            gbuf[:, 2 * DM:], wqb, preferred_element_type=jnp.float32
        ) * SCALE).astype(jnp.bfloat16)
        ls0 = []
        for h in range(HQ):
            kh = band[pl.ds(HALO, S_PER), h * DH:(h + 1) * DH]
            vh = band[pl.ds(HALO, S_PER), DM + h * DH:DM + (h + 1) * DH]
            s0 = lax.dot_general(
                q0[:, h * DH:(h + 1) * DH], kh, (((1,), (1,)), ((), ())),
                preferred_element_type=jnp.float32,
            )
            e0 = jnp.exp(s0)
            psend[:, h * DH:(h + 1) * DH] = jnp.dot(
                e0.astype(jnp.bfloat16), vh, preferred_element_type=jnp.float32
            )
            ls0.append(jnp.sum(e0, axis=1, keepdims=True))
        psend[:, DM:DM + HQ] = jnp.concatenate(ls0, axis=1)

        @pl.when(my > 0)
        def _():
            partial_send().start()

        @pl.when(my > 0)
        def _():
            halo_to_right().wait_recv()

        @pl.when(my < N_DEV - 1)
        def _():
            halo_to_left().wait_recv()

        HB = S_PER // 2
        WB = HB + 2 * HALO
        bmasks = []
        gmasks = []
        for j in (0, 1):
            qi = j * HB + lax.broadcasted_iota(jnp.int32, (HB, WB), 0)
            kb = j * HB + lax.broadcasted_iota(jnp.int32, (HB, WB), 1)
            kb_g = my * S_PER - HALO + kb
            bmasks.append(
                (kb - qi >= 0) & (kb - qi <= 2 * HALO)
                & (kb_g >= 0) & (kb_g < S_GLOB)
            )
            qg = (my * S_PER + j * HB
                  + lax.broadcasted_iota(jnp.int32, (HB, NG), 0))
            kg = lax.broadcasted_iota(jnp.int32, (HB, NG), 1)
            gmasks.append(qg > kg + HALO)

        def head_step(h, _):
            kg_ = gbuf[:, pl.ds(h * DH, DH)]
            vg_ = gbuf[:, pl.ds(DM + h * DH, DH)]
            for j in (0, 1):
                qj = qs[pl.ds(j * HB, HB), pl.ds(h * DH, DH)]
                kbj = band[pl.ds(j * HB, WB), pl.ds(h * DH, DH)]
                vbj = band[pl.ds(j * HB, WB), pl.ds(DM + h * DH, DH)]
                sb = lax.dot_general(
                    qj, kbj, (((1,), (1,)), ((), ())),
                    preferred_element_type=jnp.float32,
                ).astype(jnp.bfloat16)
                pb = jnp.where(bmasks[j], jnp.exp(sb), jnp.bfloat16(0.0))
                sg = lax.dot_general(
                    qj, kg_, (((1,), (1,)), ((), ())),
                    preferred_element_type=jnp.float32,
                ).astype(jnp.bfloat16)
                pg = jnp.where(gmasks[j], jnp.exp(sg), jnp.bfloat16(0.0))
                l = (jnp.sum(pb.astype(jnp.float32), axis=1, keepdims=True)
                     + jnp.sum(pg.astype(jnp.float32), axis=1, keepdims=True))
                ctx = (
                    jnp.dot(pb, vbj, preferred_element_type=jnp.float32)
                    + jnp.dot(pg, vg_, preferred_element_type=jnp.float32)
                ) / l
                cst[pl.ds(j * HB, HB), pl.ds(h * DH, DH)] = ctx.astype(
                    jnp.bfloat16)
            return 0

        lax.fori_loop(0, HQ, head_step, 0)

        @pl.when(my == 0)
        def _():
            for i in range(3):
                partial_recv(i).wait_recv()
            for h in range(HQ):
                num = psend[:, h * DH:(h + 1) * DH]
                den = psend[:, DM + h:DM + h + 1]
                for i in range(3):
                    num = num + precv[i, :, h * DH:(h + 1) * DH]
                    den = den + precv[i, :, DM + h:DM + h + 1]
                cst[0:NG, h * DH:(h + 1) * DH] = (num / den).astype(
                    jnp.bfloat16)

        wob = wo_ref[...].astype(jnp.bfloat16)
        out_ref[...] = jnp.dot(
            cst[...], wob, preferred_element_type=jnp.float32
        )

        @pl.when(my > 0)
        def _():
            halo_to_left().wait_send()
            partial_send().wait_send()

        @pl.when(my < N_DEV - 1)
        def _():
            halo_to_right().wait_send()

        @pl.when(my == 0)
        def _():
            for d in (1, 2, 3):
                glob_send(d).wait_send()

    out = pl.pallas_call(
        body,
        out_shape=jax.ShapeDtypeStruct((S_PER, DM), jnp.float32),
        in_specs=[
            pl.BlockSpec(memory_space=pltpu.VMEM),
            pl.BlockSpec(memory_space=pltpu.VMEM),
            pl.BlockSpec(memory_space=pltpu.VMEM),
            pl.BlockSpec(memory_space=pltpu.VMEM),
            pl.BlockSpec(memory_space=pltpu.VMEM),
        ],
        out_specs=pl.BlockSpec(memory_space=pltpu.VMEM),
        scratch_shapes=[
            pltpu.VMEM((S_BAND, 2 * DM), jnp.bfloat16),
            pltpu.VMEM((NG, 2 * DM + DM), jnp.bfloat16),
            pltpu.VMEM((S_PER, DM), jnp.bfloat16),
            pltpu.VMEM((S_PER, DM), jnp.bfloat16),
            pltpu.VMEM((S_PER, DM), jnp.bfloat16),
            pltpu.VMEM((NG, PCOLS), jnp.float32),
            pltpu.VMEM((3, NG, PCOLS), jnp.float32),
            pltpu.SemaphoreType.DMA((6,)),
            pltpu.SemaphoreType.DMA((6,)),
        ],
        compiler_params=pltpu.CompilerParams(
            collective_id=0, vmem_limit_bytes=100 * 1024 * 1024
        ),
    )(x2, Wq, K_ext, V_ext, Wo)
    return out.reshape(1, S_PER, DM)


# device time: 39033 ns/iter; 1.3927x vs baseline; 1.2523x over previous
import jax
import jax.numpy as jnp
from jax import lax
from jax.experimental import pallas as pl
from jax.experimental.pallas import tpu as pltpu

N_DEV = 4
S_PER = 1024
HQ = 8
DH = 128
DM = HQ * DH
S_GLOB = N_DEV * S_PER
HALO = 128
S_BAND = S_PER + 2 * HALO
NG = 32
PCOLS = DM + 128
SCALE = 0.08838834764831843

RI = 384
WI = RI + 2 * HALO
WE = HALO + 2 * HALO

PEERS = {0: (1, 2, 3), 1: (0, 2), 2: (0, 1, 3), 3: (0, 2)}


def kernel(x, Wq, K_ext, V_ext, Wo):
    x2 = x.reshape(S_PER, DM)

    def body(x_ref, wq_ref, k_ref, v_ref, wo_ref, out_ref,
             band, gbuf, xb, qs, cst, psend, precv, send_sems, recv_sems):
        my = lax.axis_index("i")
        left = lax.rem(my + N_DEV - 1, N_DEV)
        right = lax.rem(my + 1, N_DEV)

        barrier = pltpu.get_barrier_semaphore()
        for srcpos, dsts in PEERS.items():
            @pl.when(my == srcpos)
            def _(dsts=dsts):
                for d in dsts:
                    pl.semaphore_signal(
                        barrier, inc=1, device_id=(d,),
                        device_id_type=pl.DeviceIdType.MESH,
                    )
                for _pad in range(3 - len(dsts)):
                    pl.semaphore_signal(barrier, inc=1)

        band[pl.ds(HALO, S_PER), 0:DM] = (
            k_ref[0].reshape(S_PER, DM).astype(jnp.bfloat16))
        band[pl.ds(HALO, S_PER), DM:2 * DM] = (
            v_ref[0].reshape(S_PER, DM).astype(jnp.bfloat16))
        xb[...] = x_ref[...].astype(jnp.bfloat16)

        @pl.when(my == 0)
        def _():
            band[pl.ds(0, HALO), :] = jnp.zeros((HALO, 2 * DM), jnp.bfloat16)

        @pl.when(my == N_DEV - 1)
        def _():
            band[pl.ds(S_PER + HALO, HALO), :] = jnp.zeros(
                (HALO, 2 * DM), jnp.bfloat16
            )

        pl.semaphore_wait(barrier, 3)

        def halo_to_left():
            return pltpu.make_async_remote_copy(
                src_ref=band.at[pl.ds(HALO, HALO), :],
                dst_ref=band.at[pl.ds(S_PER + HALO, HALO), :],
                send_sem=send_sems.at[0],
                recv_sem=recv_sems.at[1],
                device_id=(left,),
                device_id_type=pl.DeviceIdType.MESH,
            )

        def halo_to_right():
            return pltpu.make_async_remote_copy(
                src_ref=band.at[pl.ds(S_PER, HALO), :],
                dst_ref=band.at[pl.ds(0, HALO), :],
                send_sem=send_sems.at[1],
                recv_sem=recv_sems.at[0],
                device_id=(right,),
                device_id_type=pl.DeviceIdType.MESH,
            )

        def glob_send(d):
            return pltpu.make_async_remote_copy(
                src_ref=gbuf,
                dst_ref=gbuf,
                send_sem=send_sems.at[2 + d],
                recv_sem=recv_sems.at[2],
                device_id=(d,),
                device_id_type=pl.DeviceIdType.MESH,
            )

        def partial_send():
            return pltpu.make_async_remote_copy(
                src_ref=psend,
                dst_ref=precv.at[my - 1],
                send_sem=send_sems.at[2],
                recv_sem=recv_sems.at[2 + my],
                device_id=(0,),
                device_id_type=pl.DeviceIdType.MESH,
            )

        def partial_recv(i):
            return pltpu.make_async_remote_copy(
                src_ref=psend,
                dst_ref=precv.at[i],
                send_sem=send_sems.at[2],
                recv_sem=recv_sems.at[3 + i],
                device_id=(0,),
                device_id_type=pl.DeviceIdType.MESH,
            )

        @pl.when(my == 0)
        def _():
            gbuf[:, 0:2 * DM] = band[pl.ds(HALO, NG), :]
            gbuf[:, 2 * DM:] = xb[0:NG, :]
            for d in (1, 2, 3):
                glob_send(d).start()

        @pl.when(my > 0)
        def _():
            halo_to_left().start()

        @pl.when(my < N_DEV - 1)
        def _():
            halo_to_right().start()

        wqb = wq_ref[...].astype(jnp.bfloat16)
        qs[...] = (jnp.dot(
            xb[...], wqb, preferred_element_type=jnp.float32
        ) * SCALE).astype(jnp.bfloat16)

        @pl.when(my > 0)
        def _():
            glob_send(1).wait_recv()

        q0 = (jnp.dot(
            gbuf[:, 2 * DM:], wqb, preferred_element_type=jnp.float32
        ) * SCALE).astype(jnp.bfloat16)
        ls0 = []
        for h in range(HQ):
            kh = band[pl.ds(HALO, S_PER), h * DH:(h + 1) * DH]
            vh = band[pl.ds(HALO, S_PER), DM + h * DH:DM + (h + 1) * DH]
            s0 = lax.dot_general(
                q0[:, h * DH:(h + 1) * DH], kh, (((1,), (1,)), ((), ())),
                preferred_element_type=jnp.float32,
            )
            e0 = jnp.exp(s0)
            psend[:, h * DH:(h + 1) * DH] = jnp.dot(
                e0.astype(jnp.bfloat16), vh, preferred_element_type=jnp.float32
            )
            ls0.append(jnp.sum(e0, axis=1, keepdims=True))
        psend[:, DM:DM + HQ] = jnp.concatenate(ls0, axis=1)

        @pl.when(my > 0)
        def _():
            partial_send().start()

        def bandblk(h, r0, c0, nrow, ncol, bmask, gmask):
            qj = qs[pl.ds(r0, nrow), pl.ds(h * DH, DH)]
            kbj = band[pl.ds(c0, ncol), pl.ds(h * DH, DH)]
            vbj = band[pl.ds(c0, ncol), pl.ds(DM + h * DH, DH)]
            kg_ = gbuf[:, pl.ds(h * DH, DH)]
            vg_ = gbuf[:, pl.ds(DM + h * DH, DH)]
            sb = lax.dot_general(
                qj, kbj, (((1,), (1,)), ((), ())),
                preferred_element_type=jnp.float32,
            ).astype(jnp.bfloat16)
            pb = jnp.where(bmask, jnp.exp(sb), jnp.bfloat16(0.0))
            sg = lax.dot_general(
                qj, kg_, (((1,), (1,)), ((), ())),
                preferred_element_type=jnp.float32,
            ).astype(jnp.bfloat16)
            pg = jnp.where(gmask, jnp.exp(sg), jnp.bfloat16(0.0))
            l = (jnp.sum(pb.astype(jnp.float32), axis=1, keepdims=True)
                 + jnp.sum(pg.astype(jnp.float32), axis=1, keepdims=True))
            ctx = (
                jnp.dot(pb, vbj, preferred_element_type=jnp.float32)
                + jnp.dot(pg, vg_, preferred_element_type=jnp.float32)
            ) / l
            cst[pl.ds(r0, nrow), pl.ds(h * DH, DH)] = ctx.astype(jnp.bfloat16)

        def gmask_rows(r0, nrow):
            qg = (my * S_PER + r0
                  + lax.broadcasted_iota(jnp.int32, (nrow, NG), 0))
            kg = lax.broadcasted_iota(jnp.int32, (nrow, NG), 1)
            return qg > kg + HALO

        i_i = lax.broadcasted_iota(jnp.int32, (RI, WI), 0)
        i_j = lax.broadcasted_iota(jnp.int32, (RI, WI), 1)
        imask = (i_j - i_i >= 0) & (i_j - i_i <= 2 * HALO)
        igm = [gmask_rows(HALO + j * RI, RI) for j in (0, 1)]

        def interior_step(h, _):
            for j in (0, 1):
                bandblk(h, HALO + j * RI, HALO + j * RI, RI, WI,
                        imask, igm[j])
            return 0

        lax.fori_loop(0, HQ, interior_step, 0)

        @pl.when(my > 0)
        def _():
            halo_to_right().wait_recv()

        @pl.when(my < N_DEV - 1)
        def _():
            halo_to_left().wait_recv()

        e_i = lax.broadcasted_iota(jnp.int32, (HALO, WE), 0)
        e_j = lax.broadcasted_iota(jnp.int32, (HALO, WE), 1)
        ein = (e_j - e_i >= 0) & (e_j - e_i <= 2 * HALO)
        tmask = ein & (my * S_PER - HALO + e_j >= 0)
        bmask = ein & (my * S_PER + S_PER - HALO - HALO + e_j < S_GLOB)
        tgm = gmask_rows(0, HALO)
        bgm = gmask_rows(S_PER - HALO, HALO)

        def edge_step(h, _):
            bandblk(h, 0, 0, HALO, WE, tmask, tgm)
            bandblk(h, S_PER - HALO, S_PER - HALO, HALO, WE, bmask, bgm)
            return 0

        lax.fori_loop(0, HQ, edge_step, 0)

        @pl.when(my == 0)
        def _():
            for i in range(3):
                partial_recv(i).wait_recv()
            for h in range(HQ):
                num = psend[:, h * DH:(h + 1) * DH]
                den = psend[:, DM + h:DM + h + 1]
                for i in range(3):
                    num = num + precv[i, :, h * DH:(h + 1) * DH]
                    den = den + precv[i, :, DM + h:DM + h + 1]
                cst[0:NG, h * DH:(h + 1) * DH] = (num / den).astype(
                    jnp.bfloat16)

        wob = wo_ref[...].astype(jnp.bfloat16)
        out_ref[...] = jnp.dot(
            cst[...], wob, preferred_element_type=jnp.float32
        )

        @pl.when(my > 0)
        def _():
            halo_to_left().wait_send()
            partial_send().wait_send()

        @pl.when(my < N_DEV - 1)
        def _():
            halo_to_right().wait_send()

        @pl.when(my == 0)
        def _():
            for d in (1, 2, 3):
                glob_send(d).wait_send()

    out = pl.pallas_call(
        body,
        out_shape=jax.ShapeDtypeStruct((S_PER, DM), jnp.float32),
        in_specs=[
            pl.BlockSpec(memory_space=pltpu.VMEM),
            pl.BlockSpec(memory_space=pltpu.VMEM),
            pl.BlockSpec(memory_space=pltpu.VMEM),
            pl.BlockSpec(memory_space=pltpu.VMEM),
            pl.BlockSpec(memory_space=pltpu.VMEM),
        ],
        out_specs=pl.BlockSpec(memory_space=pltpu.VMEM),
        scratch_shapes=[
            pltpu.VMEM((S_BAND, 2 * DM), jnp.bfloat16),
            pltpu.VMEM((NG, 2 * DM + DM), jnp.bfloat16),
            pltpu.VMEM((S_PER, DM), jnp.bfloat16),
            pltpu.VMEM((S_PER, DM), jnp.bfloat16),
            pltpu.VMEM((S_PER, DM), jnp.bfloat16),
            pltpu.VMEM((NG, PCOLS), jnp.float32),
            pltpu.VMEM((3, NG, PCOLS), jnp.float32),
            pltpu.SemaphoreType.DMA((6,)),
            pltpu.SemaphoreType.DMA((6,)),
        ],
        compiler_params=pltpu.CompilerParams(
            collective_id=0, vmem_limit_bytes=100 * 1024 * 1024
        ),
    )(x2, Wq, K_ext, V_ext, Wo)
    return out.reshape(1, S_PER, DM)
